# baseline (speedup 1.0000x reference)
import functools

import jax
import jax.numpy as jnp
from jax import lax
from jax.experimental import pallas as pl
from jax.experimental.pallas import tpu as pltpu
from jax.experimental.pallas import tpu_sc as plsc

N_NODES = 10000
N_PAD = 10240
N_EDGES = 160000
LANES = 16
NTILES = 16
NCORES = 2
CHUNK = 96
NCHUNKS = 108
WBC = 80
EDGES_PER_TILE = NCHUNKS * CHUNK
PADDED_EDGES = EDGES_PER_TILE * NTILES
FSLICE = 128
EROWS = 54
ROWS_PER_TILE = N_PAD // NTILES
WB_ROWS = 128


def _sc_segment_sum(n_slices):
    mesh = plsc.VectorSubcoreMesh(core_axis_name="c", subcore_axis_name="s")
    slices_per_core = n_slices // NCORES

    @functools.partial(
        pl.kernel,
        mesh=mesh,
        out_type=jax.ShapeDtypeStruct((n_slices * N_PAD, FSLICE), jnp.float32),
        scratch_types=[
            pltpu.VMEM((NCHUNKS * CHUNK,), jnp.int32),
            pltpu.VMEM((NCHUNKS * CHUNK,), jnp.float32),
            pltpu.VMEM((CHUNK, FSLICE), jnp.float32),
            pltpu.VMEM((CHUNK, FSLICE), jnp.float32),
            pltpu.VMEM((2, CHUNK), jnp.int32),
            pltpu.VMEM((2, CHUNK), jnp.int32),
            pltpu.VMEM_SHARED((N_PAD, FSLICE), jnp.float32),
            pltpu.SemaphoreType.DMA,
            pltpu.SemaphoreType.DMA,
        ],
    )
    def seg_sum(x_hbm, packed_hbm, w_hbm, out_hbm,
                pk_v, w_v, rows0, rows1, isrc, idst, acc, sem0, sem1):
        c = lax.axis_index("c")
        t = lax.axis_index("s")
        z16 = jnp.zeros((LANES,), jnp.float32)
        mask14 = jnp.full((LANES,), (1 << 14) - 1, jnp.int32)
        sh14 = jnp.full((LANES,), 14, jnp.int32)
        rows = (rows0, rows1)
        sems = (sem0, sem1)

        pltpu.sync_copy(packed_hbm.at[t], pk_v)
        pltpu.sync_copy(w_hbm.at[t], w_v)

        def prep(g, b, soff):
            for k in range(CHUNK // LANES):
                p = pk_v[pl.ds(g * (2 * CHUNK) + b * CHUNK + k * LANES, LANES)]
                sl = pl.ds(k * LANES, LANES)
                isrc[b, sl] = (p & mask14) + soff
                idst[b, sl] = jax.lax.shift_right_logical(p, sh14)
            pltpu.async_copy(x_hbm.at[isrc.at[b]], rows[b], sems[b])

        for si in range(slices_per_core):
            s = si * NCORES + c
            soff = jnp.full((LANES,), s * N_PAD, jnp.int32)

            def zero_row(r, _):
                for k in range(FSLICE // LANES):
                    rows0[r, pl.ds(k * LANES, LANES)] = z16
                return 0

            lax.fori_loop(0, CHUNK, zero_row, 0)
            for m in range(ROWS_PER_TILE // WBC):
                r0 = t * ROWS_PER_TILE + m * WBC
                pltpu.sync_copy(rows0.at[pl.ds(0, WBC)], acc.at[pl.ds(r0, WBC)])
            plsc.subcore_barrier()

            prep(0, 0, soff)
            prep(0, 1, soff)

            def do_pair(g, _):
                for b in range(2):
                    pltpu.make_async_copy(
                        x_hbm.at[isrc.at[b]], rows[b], sems[b]).wait()

                    def scale_group(g2, _):
                        wv16 = w_v[pl.ds(g * (2 * CHUNK) + b * CHUNK + g2 * LANES, LANES)]
                        for l in range(LANES):
                            wv = jnp.full((LANES,), wv16[l], jnp.float32)
                            e = g2 * LANES + l
                            for k in range(FSLICE // LANES):
                                sl = pl.ds(k * LANES, LANES)
                                rows[b][e, sl] = rows[b][e, sl] * wv
                        return 0

                    lax.fori_loop(0, CHUNK // LANES, scale_group, 0)

                    pltpu.sync_copy(rows[b], acc.at[idst.at[b]], add=True)

                    @pl.when(g < EROWS - 1)
                    def _():
                        prep(g + 1, b, soff)
                return 0

            lax.fori_loop(0, EROWS, do_pair, 0)
            plsc.subcore_barrier()

            for m in range(ROWS_PER_TILE // WBC):
                r0 = t * ROWS_PER_TILE + m * WBC
                pltpu.sync_copy(acc.at[pl.ds(r0, WBC)], rows0.at[pl.ds(0, WBC)])
                pltpu.sync_copy(rows0.at[pl.ds(0, WBC)],
                                out_hbm.at[pl.ds(s * N_PAD + r0, WBC)])
            plsc.subcore_barrier()

    return seg_sum


def _mlp3_body(x_ref, a_ref, wa_ref, ba_ref, wb_ref, bb_ref, wc_ref, o_ref):
    h = x_ref[...] + a_ref[...]
    t = jnp.dot(h, wa_ref[...], preferred_element_type=jnp.float32) + ba_ref[...]
    t = jnp.maximum(t, 0.0)
    t = jnp.dot(t, wb_ref[...], preferred_element_type=jnp.float32) + bb_ref[...]
    o_ref[...] = jnp.dot(t, wc_ref[...], preferred_element_type=jnp.float32)


def _mlp3(x, agg, wa, ba, wb, bb, wc, blk=2000):
    n, f = x.shape
    h = wa.shape[1]
    m = wb.shape[1]
    o = wc.shape[1]
    return pl.pallas_call(
        _mlp3_body,
        grid=(n // blk,),
        in_specs=[
            pl.BlockSpec((blk, f), lambda i: (i, 0)),
            pl.BlockSpec((blk, f), lambda i: (i, 0)),
            pl.BlockSpec((f, h), lambda i: (0, 0)),
            pl.BlockSpec((1, h), lambda i: (0, 0)),
            pl.BlockSpec((h, m), lambda i: (0, 0)),
            pl.BlockSpec((1, m), lambda i: (0, 0)),
            pl.BlockSpec((m, o), lambda i: (0, 0)),
        ],
        out_specs=pl.BlockSpec((blk, o), lambda i: (i, 0)),
        out_shape=jax.ShapeDtypeStruct((n, o), jnp.float32),
    )(x, agg, wa, ba.reshape(1, h), wb, bb.reshape(1, m), wc)


def _mlp2_body(p_ref, a_ref, bc_ref, wd_ref, bd_ref, o_ref):
    t = jnp.maximum(p_ref[...] + a_ref[...] + bc_ref[...], 0.0)
    o_ref[...] = jnp.dot(t, wd_ref[...], preferred_element_type=jnp.float32) + bd_ref[...]


def _mlp2(p, agg, bc, wd, bd, blk=2000):
    n, f = p.shape
    o = wd.shape[1]
    return pl.pallas_call(
        _mlp2_body,
        grid=(n // blk,),
        in_specs=[
            pl.BlockSpec((blk, f), lambda i: (i, 0)),
            pl.BlockSpec((blk, f), lambda i: (i, 0)),
            pl.BlockSpec((1, f), lambda i: (0, 0)),
            pl.BlockSpec((f, o), lambda i: (0, 0)),
            pl.BlockSpec((1, o), lambda i: (0, 0)),
        ],
        out_specs=pl.BlockSpec((blk, o), lambda i: (i, 0)),
        out_shape=jax.ShapeDtypeStruct((n, o), jnp.float32),
    )(p, agg, bc.reshape(1, f), wd, bd.reshape(1, o))


def _to_slices(x):
    n, f = x.shape
    s = f // FSLICE
    xp = jnp.pad(x, ((0, N_PAD - n), (0, 0)))
    return xp.reshape(N_PAD, s, FSLICE).transpose(1, 0, 2).reshape(s * N_PAD, FSLICE)


def _from_slices(x_flat, f):
    s = f // FSLICE
    return (x_flat.reshape(s, N_PAD, FSLICE).transpose(1, 0, 2)
            .reshape(N_PAD, f)[:N_NODES])


def kernel(x, edge_index, edge_attr, W1, b1, W2, b2, W3, b3, W4, b4):
    src = edge_index[0].astype(jnp.int32)
    dst = edge_index[1].astype(jnp.int32)
    w = edge_attr.astype(jnp.float32)

    pad = PADDED_EDGES - N_EDGES
    packed = src | (dst << 14)
    packedp = jnp.concatenate([packed, jnp.zeros((pad,), jnp.int32)]).reshape(
        (NTILES, EDGES_PER_TILE))
    wp = jnp.concatenate([w, jnp.zeros((pad,), jnp.float32)]).reshape(
        (NTILES, EDGES_PER_TILE))

    seg2 = _sc_segment_sum(256 // FSLICE)

    agg1 = _from_slices(seg2(_to_slices(x), packedp, wp), 256)
    p = _mlp3(x, agg1, W1, b1, W2, b2, W3)
    agg2 = _from_slices(seg2(_to_slices(p), packedp, wp), 256)
    return _mlp2(p, agg2, b3, W4, b4)


# speedup vs baseline: 1.2647x; 1.2647x over previous
import functools

import jax
import jax.numpy as jnp
from jax import lax
from jax.experimental import pallas as pl
from jax.experimental.pallas import tpu as pltpu
from jax.experimental.pallas import tpu_sc as plsc

N_NODES = 10000
N_PAD = 10240
N_EDGES = 160000
LANES = 16
NTILES = 16
NCORES = 2
CHUNK = 64
NCHUNKS = 160
EDGES_PER_TILE = NCHUNKS * CHUNK
PADDED_EDGES = EDGES_PER_TILE * NTILES
FSLICE = 128
EROWS = 80
ROWS_PER_TILE = N_PAD // NTILES
WB_ROWS = 128


def _sc_segment_sum(n_slices):
    mesh = plsc.VectorSubcoreMesh(core_axis_name="c", subcore_axis_name="s")
    slices_per_core = n_slices // NCORES

    @functools.partial(
        pl.kernel,
        mesh=mesh,
        out_type=jax.ShapeDtypeStruct((n_slices * N_PAD, FSLICE), jnp.float32),
        scratch_types=[
            pltpu.VMEM((EROWS, 2 * CHUNK), jnp.int32),
            pltpu.VMEM((EROWS, 2 * CHUNK), jnp.float32),
            pltpu.VMEM((CHUNK, FSLICE), jnp.float32),
            pltpu.VMEM((CHUNK, FSLICE), jnp.float32),
            pltpu.VMEM((2, CHUNK), jnp.int32),
            pltpu.VMEM((2, CHUNK), jnp.int32),
            pltpu.VMEM_SHARED((N_PAD, FSLICE), jnp.float32),
            pltpu.SemaphoreType.DMA,
            pltpu.SemaphoreType.DMA,
        ],
    )
    def seg_sum(x_hbm, packed_hbm, w_hbm, out_hbm,
                pk_v, w_v, rows0, rows1, isrc, idst, acc, sem0, sem1):
        c = lax.axis_index("c")
        t = lax.axis_index("s")
        z16 = jnp.zeros((LANES,), jnp.float32)
        mask14 = jnp.full((LANES,), (1 << 14) - 1, jnp.int32)
        sh14 = jnp.full((LANES,), 14, jnp.int32)
        rows = (rows0, rows1)
        sems = (sem0, sem1)

        pltpu.sync_copy(packed_hbm.at[t], pk_v)
        pltpu.sync_copy(w_hbm.at[t], w_v)

        def prep(g, b, soff):
            for k in range(CHUNK // LANES):
                p = pk_v[g, pl.ds(b * CHUNK + k * LANES, LANES)]
                sl = pl.ds(k * LANES, LANES)
                isrc[b, sl] = (p & mask14) + soff
                idst[b, sl] = jax.lax.shift_right_logical(p, sh14)
            pltpu.async_copy(x_hbm.at[isrc.at[b]], rows[b], sems[b])

        for si in range(slices_per_core):
            s = si * NCORES + c
            soff = jnp.full((LANES,), s * N_PAD, jnp.int32)

            def zero_row(r, _):
                for k in range(FSLICE // LANES):
                    rows0[r, pl.ds(k * LANES, LANES)] = z16
                return 0

            lax.fori_loop(0, CHUNK, zero_row, 0)
            for m in range(ROWS_PER_TILE // CHUNK):
                r0 = t * ROWS_PER_TILE + m * CHUNK
                pltpu.sync_copy(rows0, acc.at[pl.ds(r0, CHUNK)])
            plsc.subcore_barrier()

            prep(0, 0, soff)
            prep(0, 1, soff)

            def do_pair(g, _):
                for b in range(2):
                    pltpu.make_async_copy(
                        x_hbm.at[isrc.at[b]], rows[b], sems[b]).wait()

                    def scale_group(g2, _):
                        wv16 = w_v[g, pl.ds(b * CHUNK + g2 * LANES, LANES)]
                        for l in range(LANES):
                            wv = jnp.full((LANES,), wv16[l], jnp.float32)
                            e = g2 * LANES + l
                            for k in range(FSLICE // LANES):
                                sl = pl.ds(k * LANES, LANES)
                                rows[b][e, sl] = rows[b][e, sl] * wv
                        return 0

                    lax.fori_loop(0, CHUNK // LANES, scale_group, 0)

                    pltpu.sync_copy(rows[b], acc.at[idst.at[b]], add=True)

                    @pl.when(g < EROWS - 1)
                    def _():
                        prep(g + 1, b, soff)
                return 0

            lax.fori_loop(0, EROWS, do_pair, 0)
            plsc.subcore_barrier()

            for m in range(ROWS_PER_TILE // CHUNK):
                r0 = t * ROWS_PER_TILE + m * CHUNK
                pltpu.sync_copy(acc.at[pl.ds(r0, CHUNK)], rows0)
                pltpu.sync_copy(rows0, out_hbm.at[pl.ds(s * N_PAD + r0, CHUNK)])
            plsc.subcore_barrier()

    return seg_sum


def _mlp3_body(x_ref, a_ref, wa_ref, ba_ref, wb_ref, bb_ref, wc_ref, o_ref):
    h = x_ref[...] + a_ref[...]
    t = jnp.dot(h, wa_ref[...], preferred_element_type=jnp.float32) + ba_ref[...]
    t = jnp.maximum(t, 0.0)
    t = jnp.dot(t, wb_ref[...], preferred_element_type=jnp.float32) + bb_ref[...]
    o_ref[...] = jnp.dot(t, wc_ref[...], preferred_element_type=jnp.float32)


def _mlp3(x, agg, wa, ba, wb, bb, wc, blk=2000):
    n, f = x.shape
    h = wa.shape[1]
    m = wb.shape[1]
    o = wc.shape[1]
    return pl.pallas_call(
        _mlp3_body,
        grid=(n // blk,),
        in_specs=[
            pl.BlockSpec((blk, f), lambda i: (i, 0)),
            pl.BlockSpec((blk, f), lambda i: (i, 0)),
            pl.BlockSpec((f, h), lambda i: (0, 0)),
            pl.BlockSpec((1, h), lambda i: (0, 0)),
            pl.BlockSpec((h, m), lambda i: (0, 0)),
            pl.BlockSpec((1, m), lambda i: (0, 0)),
            pl.BlockSpec((m, o), lambda i: (0, 0)),
        ],
        out_specs=pl.BlockSpec((blk, o), lambda i: (i, 0)),
        out_shape=jax.ShapeDtypeStruct((n, o), jnp.float32),
    )(x, agg, wa, ba.reshape(1, h), wb, bb.reshape(1, m), wc)


def _mlp2_body(p_ref, a_ref, bc_ref, wd_ref, bd_ref, o_ref):
    t = jnp.maximum(p_ref[...] + a_ref[...] + bc_ref[...], 0.0)
    o_ref[...] = jnp.dot(t, wd_ref[...], preferred_element_type=jnp.float32) + bd_ref[...]


def _mlp2(p, agg, bc, wd, bd, blk=2000):
    n, f = p.shape
    o = wd.shape[1]
    return pl.pallas_call(
        _mlp2_body,
        grid=(n // blk,),
        in_specs=[
            pl.BlockSpec((blk, f), lambda i: (i, 0)),
            pl.BlockSpec((blk, f), lambda i: (i, 0)),
            pl.BlockSpec((1, f), lambda i: (0, 0)),
            pl.BlockSpec((f, o), lambda i: (0, 0)),
            pl.BlockSpec((1, o), lambda i: (0, 0)),
        ],
        out_specs=pl.BlockSpec((blk, o), lambda i: (i, 0)),
        out_shape=jax.ShapeDtypeStruct((n, o), jnp.float32),
    )(p, agg, bc.reshape(1, f), wd, bd.reshape(1, o))


def _to_slices(x):
    n, f = x.shape
    s = f // FSLICE
    xp = jnp.pad(x, ((0, N_PAD - n), (0, 0)))
    return xp.reshape(N_PAD, s, FSLICE).transpose(1, 0, 2).reshape(s * N_PAD, FSLICE)


def _from_slices(x_flat, f):
    s = f // FSLICE
    return (x_flat.reshape(s, N_PAD, FSLICE).transpose(1, 0, 2)
            .reshape(N_PAD, f)[:N_NODES])


def kernel(x, edge_index, edge_attr, W1, b1, W2, b2, W3, b3, W4, b4):
    src = edge_index[0].astype(jnp.int32)
    dst = edge_index[1].astype(jnp.int32)
    w = edge_attr.astype(jnp.float32)

    pad = PADDED_EDGES - N_EDGES
    shape3 = (NTILES, EROWS, 2 * CHUNK)
    packed = src | (dst << 14)
    packedp = jnp.concatenate([packed, jnp.zeros((pad,), jnp.int32)]).reshape(shape3)
    wp = jnp.concatenate([w, jnp.zeros((pad,), jnp.float32)]).reshape(shape3)

    seg2 = _sc_segment_sum(256 // FSLICE)

    agg1 = _from_slices(seg2(_to_slices(x), packedp, wp), 256)
    p = _mlp3(x, agg1, W1, b1, W2, b2, W3)
    agg2 = _from_slices(seg2(_to_slices(p), packedp, wp), 256)
    return _mlp2(p, agg2, b3, W4, b4)
